# Initial kernel scaffold; baseline (speedup 1.0000x reference)
import jax
import jax.numpy as jnp
from jax.experimental import pallas as pl


def kernel(mean_t, pi):
    raise NotImplementedError("write your pallas kernel here")


# speedup vs baseline: 3.9662x; 3.9662x over previous
import functools

import jax
import jax.numpy as jnp
import numpy as np
from jax import lax
from jax.experimental import pallas as pl
from jax.experimental.pallas import tpu as pltpu
from jax.experimental.pallas import tpu_sc as plsc

_N = 131072
_K = 256
_NB = 32
_NE = _NB + 1
_L = 16
_NC = 2
_NS = 16
_TILES = _NC * _NS
_ROWS_PER_TILE = _N // _TILES
_CHUNK = 128
_NCHUNKS = _ROWS_PER_TILE // _CHUNK
_GROUPS = _K // _L


_MM_BLK = 2048


def _minmax_body(x_ref, mn_ref, mx_ref):
    i = pl.program_id(0)
    m = jnp.min(x_ref[...])
    mm = jnp.max(x_ref[...])

    @pl.when(i == 0)
    def _():
        mn_ref[0, 0] = m
        mx_ref[0, 0] = mm

    @pl.when(i != 0)
    def _():
        mn_ref[0, 0] = jnp.minimum(mn_ref[0, 0], m)
        mx_ref[0, 0] = jnp.maximum(mx_ref[0, 0], mm)


def _minmax(x):
    return pl.pallas_call(
        _minmax_body,
        grid=(_N // _MM_BLK,),
        in_specs=[pl.BlockSpec((_MM_BLK, _K), lambda i: (i, 0))],
        out_specs=[
            pl.BlockSpec(memory_space=pltpu.SMEM),
            pl.BlockSpec(memory_space=pltpu.SMEM),
        ],
        out_shape=[
            jax.ShapeDtypeStruct((1, 1), jnp.float32),
            jax.ShapeDtypeStruct((1, 1), jnp.float32),
        ],
    )(x)


_sc_mesh = plsc.VectorSubcoreMesh(core_axis_name="c", subcore_axis_name="s")


@functools.partial(
    pl.kernel,
    mesh=_sc_mesh,
    compiler_params=pltpu.CompilerParams(needs_layout_passes=False),
    out_type=jax.ShapeDtypeStruct((_TILES, _NE, _K), jnp.float32),
    scratch_types=[
        pltpu.VMEM((_CHUNK, _K), jnp.float32),
        pltpu.VMEM((48,), jnp.float32),
        pltpu.VMEM((2, _L), jnp.float32),
        pltpu.VMEM((_NE, _K), jnp.float32),
    ],
)
def _hist(mean_hbm, edges_hbm, params_hbm, out_hbm, chunk_v, edges_v, params_v,
          acc_v):
    c = lax.axis_index("c")
    s = lax.axis_index("s")
    wid = s * _NC + c

    pltpu.sync_copy(edges_hbm, edges_v)
    pltpu.sync_copy(params_hbm, params_v)
    r0v = params_v[0, :]
    sv = params_v[1, :]

    zero = jnp.zeros((_L,), jnp.float32)
    ones = jnp.ones((_L,), jnp.float32)
    lane = lax.iota(jnp.int32, _L)

    def zrow(j, carry):
        for g in range(_GROUPS):
            acc_v[j, pl.ds(g * _L, _L)] = zero
        return carry

    lax.fori_loop(0, _NE, zrow, None)

    def chunk_body(ci, carry):
        row0 = wid * _ROWS_PER_TILE + ci * _CHUNK
        pltpu.sync_copy(mean_hbm.at[pl.ds(row0, _CHUNK)], chunk_v)

        def row_body(r, rc):
            for g in range(_GROUPS):
                x = chunk_v[r, pl.ds(g * _L, _L)]
                t = (x - r0v) * sv
                b = t.astype(jnp.int32)
                eb = plsc.load_gather(edges_v, [b])
                b = jnp.where(x < eb, b - 1, b)
                b = jnp.maximum(b, 0)
                bp = jnp.minimum(b + 1, _NB)
                e2 = plsc.load_gather(edges_v, [bp])
                b = jnp.where(x >= e2, bp, b)
                col = lane + g * _L
                plsc.addupdate_scatter(acc_v, [b, col], ones)
            return rc

        lax.fori_loop(0, _CHUNK, row_body, None)
        return carry

    lax.fori_loop(0, _NCHUNKS, chunk_body, None)
    pltpu.sync_copy(acc_v, out_hbm.at[wid])


_INV_LN2 = np.float32(1.0 / np.log(2.0))


def _entropy_body(parts_ref, pi_ref, out_ref):
    counts = jnp.sum(parts_ref[...], axis=0)
    counts = counts[:_NB, :]
    d = counts * np.float32(1.0 / _N)
    ent = jnp.sum(-d * jnp.log(d + np.float32(1e-7)), axis=0, keepdims=True)
    out_ref[0, 0] = jnp.sum((1.0 - pi_ref[...]) * ent) * _INV_LN2


def _entropy(parts, pi):
    return pl.pallas_call(
        _entropy_body,
        out_specs=pl.BlockSpec(memory_space=pltpu.SMEM),
        out_shape=jax.ShapeDtypeStruct((1, 1), jnp.float32),
    )(parts, pi)


def kernel(mean_t, pi):
    mn, mx = _minmax(mean_t)
    r0 = mn[0, 0]
    r1 = mx[0, 0]
    edges = jnp.linspace(r0, r1, _NE).astype(jnp.float32)
    scale = jnp.where(r1 > r0, _NB / (r1 - r0), 0.0).astype(jnp.float32)
    edges_pad = jnp.concatenate(
        [edges, jnp.full((48 - _NE,), edges[-1], jnp.float32)])
    params = jnp.stack([
        jnp.full((_L,), r0, jnp.float32),
        jnp.full((_L,), scale, jnp.float32),
    ])
    parts = _hist(mean_t, edges_pad, params)
    ixt = _entropy(parts, pi)[0]
    return jnp.where(r1 > r0, ixt, jnp.zeros((1,), jnp.float32))


# speedup vs baseline: 21.1240x; 5.3259x over previous
import functools

import jax
import jax.numpy as jnp
import numpy as np
from jax import lax
from jax.experimental import pallas as pl
from jax.experimental.pallas import tpu as pltpu
from jax.experimental.pallas import tpu_sc as plsc

_N = 131072
_K = 256
_NB = 32
_NE = _NB + 1
_L = 16
_NC = 2
_NS = 16
_TILES = _NC * _NS
_ROWS_PER_TILE = _N // _TILES
_CHUNK = 128
_NCHUNKS = _ROWS_PER_TILE // _CHUNK
_GROUPS = _K // _L


_MM_BLK = 2048


def _minmax_body(x_ref, mn_ref, mx_ref):
    i = pl.program_id(0)
    m = jnp.min(x_ref[...])
    mm = jnp.max(x_ref[...])

    @pl.when(i == 0)
    def _():
        mn_ref[0, 0] = m
        mx_ref[0, 0] = mm

    @pl.when(i != 0)
    def _():
        mn_ref[0, 0] = jnp.minimum(mn_ref[0, 0], m)
        mx_ref[0, 0] = jnp.maximum(mx_ref[0, 0], mm)


def _minmax(x):
    return pl.pallas_call(
        _minmax_body,
        grid=(_N // _MM_BLK,),
        in_specs=[pl.BlockSpec((_MM_BLK, _K), lambda i: (i, 0))],
        out_specs=[
            pl.BlockSpec(memory_space=pltpu.SMEM),
            pl.BlockSpec(memory_space=pltpu.SMEM),
        ],
        out_shape=[
            jax.ShapeDtypeStruct((1, 1), jnp.float32),
            jax.ShapeDtypeStruct((1, 1), jnp.float32),
        ],
    )(x)


_sc_mesh = plsc.VectorSubcoreMesh(core_axis_name="c", subcore_axis_name="s")


@functools.partial(
    pl.kernel,
    mesh=_sc_mesh,
    compiler_params=pltpu.CompilerParams(needs_layout_passes=False),
    out_type=jax.ShapeDtypeStruct((_TILES, _NE, _K), jnp.float32),
    scratch_types=[
        pltpu.VMEM((_CHUNK, _K), jnp.float32),
        pltpu.VMEM((_CHUNK, _K), jnp.float32),
        pltpu.VMEM((48,), jnp.float32),
        pltpu.VMEM((2, _L), jnp.float32),
        pltpu.VMEM((_NE, _K), jnp.float32),
        pltpu.SemaphoreType.DMA,
        pltpu.SemaphoreType.DMA,
    ],
)
def _hist(mean_hbm, edges_hbm, params_hbm, out_hbm, chunk0, chunk1, edges_v,
          params_v, acc_v, sem0, sem1):
    c = lax.axis_index("c")
    s = lax.axis_index("s")
    wid = s * _NC + c

    pltpu.sync_copy(edges_hbm, edges_v)
    pltpu.sync_copy(params_hbm, params_v)
    r0v = params_v[0, :]
    sv = params_v[1, :]

    zero = jnp.zeros((_L,), jnp.float32)
    ones = jnp.ones((_L,), jnp.float32)
    lane = lax.iota(jnp.int32, _L)

    def zrow(j, carry):
        for g in range(_GROUPS):
            acc_v[j, pl.ds(g * _L, _L)] = zero
        return carry

    lax.fori_loop(0, _NE, zrow, None)

    def copy(ci, buf, sem):
        row0 = wid * _ROWS_PER_TILE + ci * _CHUNK
        return pltpu.make_async_copy(
            mean_hbm.at[pl.ds(row0, _CHUNK)], buf, sem)

    def process(buf):
        @plsc.parallel_loop(0, _CHUNK * _GROUPS, unroll=8)
        def _(v):
            r = v // _GROUPS
            g = v % _GROUPS
            x = buf[r, pl.ds(g * _L, _L)]
            t = (x - r0v) * sv
            b = t.astype(jnp.int32)
            eb = plsc.load_gather(edges_v, [b])
            b = jnp.where(x < eb, b - 1, b)
            b = jnp.maximum(b, 0)
            bp = jnp.minimum(b + 1, _NB)
            e2 = plsc.load_gather(edges_v, [bp])
            b = jnp.where(x >= e2, bp, b)
            col = lane + g * _L
            plsc.addupdate_scatter(acc_v, [b, col], ones)

    copy(0, chunk0, sem0).start()
    copy(1, chunk1, sem1).start()

    def outer(ci2, carry):
        for b_, (buf, sem) in enumerate(((chunk0, sem0), (chunk1, sem1))):
            ci = ci2 * 2 + b_
            copy(ci, buf, sem).wait()
            process(buf)

            @pl.when(ci + 2 < _NCHUNKS)
            def _():
                copy(ci + 2, buf, sem).start()

        return carry

    lax.fori_loop(0, _NCHUNKS // 2, outer, None)
    pltpu.sync_copy(acc_v, out_hbm.at[wid])


_INV_LN2 = np.float32(1.0 / np.log(2.0))


def _entropy_body(parts_ref, pi_ref, out_ref):
    counts = jnp.sum(parts_ref[...], axis=0)
    counts = counts[:_NB, :]
    d = counts * np.float32(1.0 / _N)
    ent = jnp.sum(-d * jnp.log(d + np.float32(1e-7)), axis=0, keepdims=True)
    out_ref[0, 0] = jnp.sum((1.0 - pi_ref[...]) * ent) * _INV_LN2


def _entropy(parts, pi):
    return pl.pallas_call(
        _entropy_body,
        out_specs=pl.BlockSpec(memory_space=pltpu.SMEM),
        out_shape=jax.ShapeDtypeStruct((1, 1), jnp.float32),
    )(parts, pi)


def kernel(mean_t, pi):
    mn, mx = _minmax(mean_t)
    r0 = mn[0, 0]
    r1 = mx[0, 0]
    edges = jnp.linspace(r0, r1, _NE).astype(jnp.float32)
    scale = jnp.where(r1 > r0, _NB / (r1 - r0), 0.0).astype(jnp.float32)
    edges_pad = jnp.concatenate(
        [edges, jnp.full((48 - _NE,), edges[-1], jnp.float32)])
    params = jnp.stack([
        jnp.full((_L,), r0, jnp.float32),
        jnp.full((_L,), scale, jnp.float32),
    ])
    parts = _hist(mean_t, edges_pad, params)
    ixt = _entropy(parts, pi)[0]
    return jnp.where(r1 > r0, ixt, jnp.zeros((1,), jnp.float32))


# speedup vs baseline: 24.2375x; 1.1474x over previous
import functools

import jax
import jax.numpy as jnp
import numpy as np
from jax import lax
from jax.experimental import pallas as pl
from jax.experimental.pallas import tpu as pltpu
from jax.experimental.pallas import tpu_sc as plsc

_N = 131072
_K = 256
_NB = 32
_NE = _NB + 1
_L = 16
_NC = 2
_NS = 16
_TILES = _NC * _NS
_ROWS_PER_TILE = _N // _TILES
_CHUNK = 128
_NCHUNKS = _ROWS_PER_TILE // _CHUNK
_GROUPS = _K // _L


_MM_BLK = 2048


def _minmax_body(x_ref, mn_ref, mx_ref):
    i = pl.program_id(0)
    m = jnp.min(x_ref[...])
    mm = jnp.max(x_ref[...])

    @pl.when(i == 0)
    def _():
        mn_ref[0, 0] = m
        mx_ref[0, 0] = mm

    @pl.when(i != 0)
    def _():
        mn_ref[0, 0] = jnp.minimum(mn_ref[0, 0], m)
        mx_ref[0, 0] = jnp.maximum(mx_ref[0, 0], mm)


def _minmax(x):
    return pl.pallas_call(
        _minmax_body,
        grid=(_N // _MM_BLK,),
        in_specs=[pl.BlockSpec((_MM_BLK, _K), lambda i: (i, 0))],
        out_specs=[
            pl.BlockSpec(memory_space=pltpu.SMEM),
            pl.BlockSpec(memory_space=pltpu.SMEM),
        ],
        out_shape=[
            jax.ShapeDtypeStruct((1, 1), jnp.float32),
            jax.ShapeDtypeStruct((1, 1), jnp.float32),
        ],
    )(x)


_sc_mesh = plsc.VectorSubcoreMesh(core_axis_name="c", subcore_axis_name="s")


@functools.partial(
    pl.kernel,
    mesh=_sc_mesh,
    compiler_params=pltpu.CompilerParams(needs_layout_passes=False),
    out_type=jax.ShapeDtypeStruct((_TILES, _NE, _K), jnp.float32),
    scratch_types=[
        pltpu.VMEM((_CHUNK, _K), jnp.float32),
        pltpu.VMEM((_CHUNK, _K), jnp.float32),
        pltpu.VMEM((48 * _L,), jnp.float32),
        pltpu.VMEM((2, _L), jnp.float32),
        pltpu.VMEM((_NE, _K), jnp.float32),
        pltpu.SemaphoreType.DMA,
        pltpu.SemaphoreType.DMA,
    ],
)
def _hist(mean_hbm, edges_hbm, params_hbm, out_hbm, chunk0, chunk1, edges_v,
          params_v, acc_v, sem0, sem1):
    c = lax.axis_index("c")
    s = lax.axis_index("s")
    wid = s * _NC + c

    pltpu.sync_copy(edges_hbm, edges_v)
    pltpu.sync_copy(params_hbm, params_v)
    r0v = params_v[0, :]
    sv = params_v[1, :]

    zero = jnp.zeros((_L,), jnp.float32)
    ones = jnp.ones((_L,), jnp.float32)
    lane = lax.iota(jnp.int32, _L)

    def zrow(j, carry):
        for g in range(_GROUPS):
            acc_v[j, pl.ds(g * _L, _L)] = zero
        return carry

    lax.fori_loop(0, _NE, zrow, None)

    def copy(ci, buf, sem):
        row0 = wid * _ROWS_PER_TILE + ci * _CHUNK
        return pltpu.make_async_copy(
            mean_hbm.at[pl.ds(row0, _CHUNK)], buf, sem)

    def process(buf):
        @plsc.parallel_loop(0, _CHUNK * _GROUPS, unroll=8)
        def _(v):
            r = v // _GROUPS
            g = v % _GROUPS
            x = buf[r, pl.ds(g * _L, _L)]
            t = (x - r0v) * sv + jnp.float32(1e-4)
            b = t.astype(jnp.int32)
            eb = plsc.load_gather(edges_v, [b * _L + lane])
            b = jnp.where(x < eb, b - 1, b)
            col = lane + g * _L
            plsc.addupdate_scatter(acc_v, [b, col], ones)

    copy(0, chunk0, sem0).start()
    copy(1, chunk1, sem1).start()

    def outer(ci2, carry):
        for b_, (buf, sem) in enumerate(((chunk0, sem0), (chunk1, sem1))):
            ci = ci2 * 2 + b_
            copy(ci, buf, sem).wait()
            process(buf)

            @pl.when(ci + 2 < _NCHUNKS)
            def _():
                copy(ci + 2, buf, sem).start()

        return carry

    lax.fori_loop(0, _NCHUNKS // 2, outer, None)
    pltpu.sync_copy(acc_v, out_hbm.at[wid])


_INV_LN2 = np.float32(1.0 / np.log(2.0))


def _entropy_body(parts_ref, pi_ref, out_ref):
    counts = jnp.sum(parts_ref[...], axis=0)
    counts = counts[:_NB, :]
    d = counts * np.float32(1.0 / _N)
    ent = jnp.sum(-d * jnp.log(d + np.float32(1e-7)), axis=0, keepdims=True)
    out_ref[0, 0] = jnp.sum((1.0 - pi_ref[...]) * ent) * _INV_LN2


def _entropy(parts, pi):
    return pl.pallas_call(
        _entropy_body,
        out_specs=pl.BlockSpec(memory_space=pltpu.SMEM),
        out_shape=jax.ShapeDtypeStruct((1, 1), jnp.float32),
    )(parts, pi)


def kernel(mean_t, pi):
    mn, mx = _minmax(mean_t)
    r0 = mn[0, 0]
    r1 = mx[0, 0]
    edges = jnp.linspace(r0, r1, _NE).astype(jnp.float32)
    scale = jnp.where(r1 > r0, _NB / (r1 - r0), 0.0).astype(jnp.float32)
    edges_pad = jnp.concatenate(
        [edges, jnp.full((48 - _NE,), edges[-1], jnp.float32)])
    edges_rep = jnp.tile(edges_pad[:, None], (1, _L)).reshape(-1)
    params = jnp.stack([
        jnp.full((_L,), r0, jnp.float32),
        jnp.full((_L,), scale, jnp.float32),
    ])
    parts = _hist(mean_t, edges_rep, params)
    ixt = _entropy(parts, pi)[0]
    return jnp.where(r1 > r0, ixt, jnp.zeros((1,), jnp.float32))


# speedup vs baseline: 31.2041x; 1.2874x over previous
import functools

import jax
import jax.numpy as jnp
import numpy as np
from jax import lax
from jax.experimental import pallas as pl
from jax.experimental.pallas import tpu as pltpu
from jax.experimental.pallas import tpu_sc as plsc

_N = 131072
_K = 256
_NB = 32
_NE = _NB + 1
_L = 16
_NC = 2
_NS = 16
_TILES = _NC * _NS
_ROWS_PER_TILE = _N // _TILES
_CHUNK = 128
_NCHUNKS = _ROWS_PER_TILE // _CHUNK
_GROUPS = _K // _L


_MM_BLK = 2048


def _minmax_body(x_ref, mn_ref, mx_ref):
    i = pl.program_id(0)
    m = jnp.min(x_ref[...])
    mm = jnp.max(x_ref[...])

    @pl.when(i == 0)
    def _():
        mn_ref[0, 0] = m
        mx_ref[0, 0] = mm

    @pl.when(i != 0)
    def _():
        mn_ref[0, 0] = jnp.minimum(mn_ref[0, 0], m)
        mx_ref[0, 0] = jnp.maximum(mx_ref[0, 0], mm)


def _minmax(x):
    return pl.pallas_call(
        _minmax_body,
        grid=(_N // _MM_BLK,),
        in_specs=[pl.BlockSpec((_MM_BLK, _K), lambda i: (i, 0))],
        out_specs=[
            pl.BlockSpec(memory_space=pltpu.SMEM),
            pl.BlockSpec(memory_space=pltpu.SMEM),
        ],
        out_shape=[
            jax.ShapeDtypeStruct((1, 1), jnp.float32),
            jax.ShapeDtypeStruct((1, 1), jnp.float32),
        ],
    )(x)


_sc_mesh = plsc.VectorSubcoreMesh(core_axis_name="c", subcore_axis_name="s")


@functools.partial(
    pl.kernel,
    mesh=_sc_mesh,
    compiler_params=pltpu.CompilerParams(needs_layout_passes=False),
    out_type=jax.ShapeDtypeStruct((_TILES, _NE, _K), jnp.float32),
    scratch_types=[
        pltpu.VMEM((_CHUNK, _K), jnp.float32),
        pltpu.VMEM((_CHUNK, _K), jnp.float32),
        pltpu.VMEM((48 * _L,), jnp.float32),
        pltpu.VMEM((2, _L), jnp.float32),
        pltpu.VMEM((_NE, _K), jnp.float32),
        pltpu.SemaphoreType.DMA,
        pltpu.SemaphoreType.DMA,
    ],
)
def _hist(mean_hbm, edges_hbm, params_hbm, out_hbm, chunk0, chunk1, edges_v,
          params_v, acc_v, sem0, sem1):
    c = lax.axis_index("c")
    s = lax.axis_index("s")
    wid = s * _NC + c

    pltpu.sync_copy(edges_hbm, edges_v)
    pltpu.sync_copy(params_hbm, params_v)
    r0v = params_v[0, :]
    sv = params_v[1, :]

    zero = jnp.zeros((_L,), jnp.float32)
    ones = jnp.ones((_L,), jnp.float32)
    lane = lax.iota(jnp.int32, _L)

    def zrow(j, carry):
        for g in range(_GROUPS):
            acc_v[j, pl.ds(g * _L, _L)] = zero
        return carry

    lax.fori_loop(0, _NE, zrow, None)

    def copy(ci, buf, sem):
        row0 = wid * _ROWS_PER_TILE + ci * _CHUNK
        return pltpu.make_async_copy(
            mean_hbm.at[pl.ds(row0, _CHUNK)], buf, sem)

    def process(buf):
        @plsc.parallel_loop(0, _CHUNK * _GROUPS, unroll=8)
        def _(v):
            r = v // _GROUPS
            g = v % _GROUPS
            x = buf[r, pl.ds(g * _L, _L)]
            t = (x - r0v) * sv
            b = t.astype(jnp.int32)
            col = lane + g * _L
            plsc.addupdate_scatter(acc_v, [b, col], ones)

    copy(0, chunk0, sem0).start()
    copy(1, chunk1, sem1).start()

    def outer(ci2, carry):
        for b_, (buf, sem) in enumerate(((chunk0, sem0), (chunk1, sem1))):
            ci = ci2 * 2 + b_
            copy(ci, buf, sem).wait()
            process(buf)

            @pl.when(ci + 2 < _NCHUNKS)
            def _():
                copy(ci + 2, buf, sem).start()

        return carry

    lax.fori_loop(0, _NCHUNKS // 2, outer, None)
    pltpu.sync_copy(acc_v, out_hbm.at[wid])


_INV_LN2 = np.float32(1.0 / np.log(2.0))


def _entropy_body(parts_ref, pi_ref, out_ref):
    counts = jnp.sum(parts_ref[...], axis=0)
    counts = counts[:_NB, :]
    d = counts * np.float32(1.0 / _N)
    ent = jnp.sum(-d * jnp.log(d + np.float32(1e-7)), axis=0, keepdims=True)
    out_ref[0, 0] = jnp.sum((1.0 - pi_ref[...]) * ent) * _INV_LN2


def _entropy(parts, pi):
    return pl.pallas_call(
        _entropy_body,
        out_specs=pl.BlockSpec(memory_space=pltpu.SMEM),
        out_shape=jax.ShapeDtypeStruct((1, 1), jnp.float32),
    )(parts, pi)


def kernel(mean_t, pi):
    mn, mx = _minmax(mean_t)
    r0 = mn[0, 0]
    r1 = mx[0, 0]
    edges = jnp.linspace(r0, r1, _NE).astype(jnp.float32)
    scale = jnp.where(r1 > r0, _NB / (r1 - r0), 0.0).astype(jnp.float32)
    edges_pad = jnp.concatenate(
        [edges, jnp.full((48 - _NE,), edges[-1], jnp.float32)])
    edges_rep = jnp.tile(edges_pad[:, None], (1, _L)).reshape(-1)
    params = jnp.stack([
        jnp.full((_L,), r0, jnp.float32),
        jnp.full((_L,), scale, jnp.float32),
    ])
    parts = _hist(mean_t, edges_rep, params)
    ixt = _entropy(parts, pi)[0]
    return jnp.where(r1 > r0, ixt, jnp.zeros((1,), jnp.float32))
